# baseline (speedup 1.0000x reference)
import functools

import jax
import jax.numpy as jnp
from jax import lax
from jax.experimental import pallas as pl
from jax.experimental.pallas import tpu as pltpu
from jax.experimental.pallas import tpu_sc as plsc

N = 10000
E = 320000
D = 128
NC, NS = 2, 16
NW = NC * NS
RANGE = 320
NPAD = NW * RANGE
SENT = RANGE
ACC_ROWS = RANGE + 8
EB = 128
EP = NW * 10240
EROWS = EP // 128
SROWS = 80
PAD_DST = 10560
OUT_SZ = E + NW * EB + EB + SROWS * 128
DUMP = E + NW * EB
ROW_BLK = 1000

@functools.cache
def _mesh():
    return plsc.VectorSubcoreMesh(
        core_axis_name="c", subcore_axis_name="s", num_cores=NC, num_subcores=NS)


def _wid():
    return lax.axis_index("s") * NC + lax.axis_index("c")


def _rank_a_body(dst_ref, src_ref, rank_ref, rc_ref, dloc_ref):
    k = pl.program_id(0)
    dst = dst_ref[...]
    bucket = dst // RANGE
    lane32 = jax.lax.broadcasted_iota(jnp.int32, (1, 32), 1)
    m = (bucket == k)
    mf = jnp.where(m, 1.0, 0.0).astype(jnp.float32)
    io0 = jax.lax.broadcasted_iota(jnp.int32, (128, 128), 0)
    io1 = jax.lax.broadcasted_iota(jnp.int32, (128, 128), 1)
    tri = jnp.where(io0 <= io1, 1.0, 0.0).astype(jnp.float32)
    c = jnp.dot(mf, tri, preferred_element_type=jnp.float32)

    @pl.when(k == 0)
    def _():
        rank_ref[...] = jnp.zeros((EROWS, 128), jnp.float32)
        rc_ref[...] = jnp.zeros((EROWS, 32), jnp.float32)
        dloc_ref[...] = (dst - bucket * RANGE) * 16384 + src_ref[...]

    rank_ref[...] += jnp.where(m, c - mf, 0.0)
    rc_ref[...] += jnp.where(lane32 == k, c[:, 127:128], 0.0)


def _rank_b_body(rc_ref, exrow_ref, base_ref, spos_ref, meta_ref):
    rc_all = rc_ref[...]
    lane = jax.lax.broadcasted_iota(jnp.int32, (1, 128), 1)
    rc_cum = rc_all
    s = 1
    while s < EROWS:
        rc_cum = rc_cum + jnp.concatenate(
            [jnp.zeros((s, 32), jnp.float32), rc_cum[:-s, :]], axis=0)
        s *= 2
    totals = rc_cum[EROWS - 1:EROWS, :]
    tot_i = totals.astype(jnp.int32)
    nblk = (tot_i + (EB - 1)) // EB
    padded = nblk * EB
    base = padded
    for s in (1, 2, 4, 8, 16):
        base = base + jnp.concatenate(
            [jnp.zeros((1, s), jnp.int32), base[:, :-s]], axis=1)
    base = base - padded
    exrow_ref[...] = rc_cum - rc_all + base.astype(jnp.float32)
    base_ref[...] = base

    srows = []
    mrows = []
    zl14 = jnp.zeros((1, 14), jnp.int32)
    for k in range(NW):
        bk = base[:, k:k + 1]
        tk = tot_i[:, k:k + 1]
        pk = padded[:, k:k + 1]
        sp = jnp.where(lane < pk - tk, bk + tk + lane, DUMP + lane)
        srows.append(sp)
        mrows.append(jnp.concatenate([bk, nblk[:, k:k + 1], zl14], axis=1))
    spos_ref[...] = jnp.concatenate(srows, axis=0)
    meta_ref[...] = jnp.concatenate(mrows, axis=0)


def _rank_c_body(dst_ref, rank_ref, exrow_ref, pos_ref, acc_ref):
    k = pl.program_id(0)
    dst = dst_ref[...]
    bucket = dst // RANGE

    @pl.when(k == 0)
    def _():
        acc_ref[...] = rank_ref[...]

    lane32 = jax.lax.broadcasted_iota(jnp.int32, (1, 32), 1)
    exsel = jnp.sum(jnp.where(lane32 == k, exrow_ref[...], 0.0),
                    axis=1, keepdims=True)
    acc_ref[...] += jnp.where(bucket == k, exsel, 0.0)

    @pl.when(k == NW - 1)
    def _():
        lane = jax.lax.broadcasted_iota(jnp.int32, (1, 128), 1)
        pos = acc_ref[...].astype(jnp.int32)
        pos_ref[...] = jnp.where(dst >= N, DUMP + (lane % EB), pos)


def _rank(dstp, srcp):
    rank, rc_all, dloc = pl.pallas_call(
        _rank_a_body,
        grid=(NW,),
        in_specs=[
            pl.BlockSpec((EROWS, 128), lambda k: (0, 0)),
            pl.BlockSpec((EROWS, 128), lambda k: (0, 0)),
        ],
        out_specs=[
            pl.BlockSpec((EROWS, 128), lambda k: (0, 0)),
            pl.BlockSpec((EROWS, 32), lambda k: (0, 0)),
            pl.BlockSpec((EROWS, 128), lambda k: (0, 0)),
        ],
        out_shape=[
            jax.ShapeDtypeStruct((EROWS, 128), jnp.float32),
            jax.ShapeDtypeStruct((EROWS, 32), jnp.float32),
            jax.ShapeDtypeStruct((EROWS, 128), jnp.int32),
        ],
    )(dstp, srcp)
    exrow, base, spos, meta = pl.pallas_call(
        _rank_b_body,
        grid=(1,),
        in_specs=[pl.BlockSpec((EROWS, 32), lambda i: (0, 0))],
        out_specs=[
            pl.BlockSpec((EROWS, 32), lambda i: (0, 0)),
            pl.BlockSpec((1, 32), lambda i: (0, 0)),
            pl.BlockSpec((NW, 128), lambda i: (0, 0)),
            pl.BlockSpec((NW, 16), lambda i: (0, 0)),
        ],
        out_shape=[
            jax.ShapeDtypeStruct((EROWS, 32), jnp.float32),
            jax.ShapeDtypeStruct((1, 32), jnp.int32),
            jax.ShapeDtypeStruct((NW, 128), jnp.int32),
            jax.ShapeDtypeStruct((NW, 16), jnp.int32),
        ],
    )(rc_all)
    pos = pl.pallas_call(
        _rank_c_body,
        grid=(NW,),
        in_specs=[
            pl.BlockSpec((EROWS, 128), lambda k: (0, 0)),
            pl.BlockSpec((EROWS, 128), lambda k: (0, 0)),
            pl.BlockSpec((EROWS, 32), lambda k: (0, 0)),
        ],
        out_specs=pl.BlockSpec((EROWS, 128), lambda k: (0, 0)),
        out_shape=jax.ShapeDtypeStruct((EROWS, 128), jnp.int32),
        scratch_shapes=[pltpu.VMEM((EROWS, 128), jnp.float32)],
    )(dstp, rank, exrow)
    return pos, dloc, spos, meta


def _scatter_body(pos2, vcomb2, spos_hbm, ecomb_hbm,
                  idxv, valv, sposv, dlbuf, sem):
    wid = _wid()
    pltpu.sync_copy(pos2.at[wid], idxv)
    pltpu.sync_copy(vcomb2.at[wid], valv)

    sent16 = jnp.zeros((16,), jnp.int32) + SENT * 16384
    for j in range(8):
        dlbuf[pl.ds(j * 16, 16)] = sent16
    pltpu.sync_copy(spos_hbm.at[wid], sposv)

    cp1 = pltpu.make_async_copy(valv, ecomb_hbm.at[idxv], sem)
    cp1.start()
    cp2 = pltpu.make_async_copy(dlbuf, ecomb_hbm.at[sposv], sem)
    cp2.start()
    cp1.wait()
    cp2.wait()


@functools.cache
def _scatter_kernel():
  return functools.partial(
    pl.kernel,
    out_type=jax.ShapeDtypeStruct((OUT_SZ,), jnp.int32),
    mesh=_mesh(),
    scratch_types=[
        pltpu.VMEM((SROWS * 128,), jnp.int32),
        pltpu.VMEM((SROWS * 128,), jnp.int32),
        pltpu.VMEM((128,), jnp.int32),
        pltpu.VMEM((128,), jnp.int32),
        pltpu.SemaphoreType.DMA,
    ],
  )(_scatter_body)


def _segmax_body(hp_hbm, ecomb_hbm, meta_hbm, agg_hbm,
                 acc, rows, ev, sidx, cbuf, sem0, sem1):
    wid = _wid()
    zero16 = jnp.zeros((16,), jnp.float32)

    def zrow(r, _):
        for f in range(8):
            acc[r, pl.ds(f * 16, 16)] = zero16
        return 0

    lax.fori_loop(0, ACC_ROWS, zrow, 0)

    pltpu.sync_copy(meta_hbm.at[wid], cbuf)
    cv = cbuf[pl.ds(0, 16)]
    base = cv[0]
    nblk = cv[1]

    def start_gather(b, slot, sem):
        pltpu.make_async_copy(
            hp_hbm.at[sidx.at[pl.ds(b * EB, EB)]], rows.at[slot], sem).start()

    def wait_gather(b, slot, sem):
        pltpu.make_async_copy(
            hp_hbm.at[sidx.at[pl.ds(b * EB, EB)]], rows.at[slot], sem).wait()

    def process(b, slot):
        def grp(g, _):
            dvec = ev[pl.ds(b * EB + g * 16, 16)] >> 14
            for j in range(16):
                e = g * 16 + j
                d = dvec[j]
                ar = acc.at[d]
                rr = rows.at[slot, e]
                for f in range(8):
                    sl = pl.ds(f * 16, 16)
                    ar[sl] = jnp.maximum(ar[sl], rr[sl])
            return 0

        lax.fori_loop(0, EB // 16, grp, 0)

    nsb = (nblk + SROWS - 1) // SROWS

    def sblk(sb, _):
        sb_base = pl.multiple_of(base + sb * (SROWS * EB), 8)
        pltpu.sync_copy(ecomb_hbm.at[pl.ds(sb_base, SROWS * EB)], ev)

        def mkidx(r, _):
            for g in range(8):
                o = pl.ds(r * EB + g * 16, 16)
                sidx[o] = ev[o] & 16383
            return 0

        lax.fori_loop(0, SROWS, mkidx, 0)
        lim = jnp.minimum(SROWS, nblk - sb * SROWS)
        start_gather(0, 0, sem0)

        def blk(b, _):
            @pl.when(b % 2 == 0)
            def _():
                @pl.when(b + 1 < lim)
                def _():
                    start_gather(b + 1, 1, sem1)
                wait_gather(b, 0, sem0)
                process(b, 0)

            @pl.when(b % 2 == 1)
            def _():
                @pl.when(b + 1 < lim)
                def _():
                    start_gather(b + 1, 0, sem0)
                wait_gather(b, 1, sem1)
                process(b, 1)

            return 0

        lax.fori_loop(0, lim, blk, 0)
        return 0

    lax.fori_loop(0, nsb, sblk, 0)

    pltpu.sync_copy(acc.at[pl.ds(0, RANGE)], agg_hbm.at[pl.ds(wid * RANGE, RANGE)])


@functools.cache
def _segmax_kernel():
  return functools.partial(
    pl.kernel,
    out_type=jax.ShapeDtypeStruct((NPAD, D), jnp.float32),
    mesh=_mesh(),
    scratch_types=[
        pltpu.VMEM((ACC_ROWS, D), jnp.float32),
        pltpu.VMEM((2, EB, D), jnp.float32),
        pltpu.VMEM((SROWS * EB,), jnp.int32),
        pltpu.VMEM((SROWS * EB,), jnp.int32),
        pltpu.VMEM((16,), jnp.int32),
        pltpu.SemaphoreType.DMA,
        pltpu.SemaphoreType.DMA,
    ],
  )(_segmax_body)


def _pool_body(h_ref, w_ref, b_ref, o_ref):
    acc = jnp.dot(h_ref[...], w_ref[...], preferred_element_type=jnp.float32)
    o_ref[...] = jnp.maximum(acc + b_ref[...], 0.0)


def _pool_matmul(h, W, b):
    grid = (N // ROW_BLK,)
    return pl.pallas_call(
        _pool_body,
        grid=grid,
        in_specs=[
            pl.BlockSpec((ROW_BLK, D), lambda i: (i, 0)),
            pl.BlockSpec((D, D), lambda i: (0, 0)),
            pl.BlockSpec((1, D), lambda i: (0, 0)),
        ],
        out_specs=pl.BlockSpec((ROW_BLK, D), lambda i: (i, 0)),
        out_shape=jax.ShapeDtypeStruct((N, D), jnp.float32),
    )(h, W, b.reshape(1, D))


def _out_body(apply_act, h_ref, agg_ref, ws_ref, wn_ref, b_ref, o_ref):
    acc = jnp.dot(h_ref[...], ws_ref[...], preferred_element_type=jnp.float32)
    acc += jnp.dot(agg_ref[...], wn_ref[...], preferred_element_type=jnp.float32)
    acc += b_ref[...]
    if apply_act:
        acc = jnp.where(acc > 0.0, acc, jnp.exp(jnp.minimum(acc, 0.0)) - 1.0)
    o_ref[...] = acc


def _out_matmul(h, agg_pad, W_self, W_neigh, b_self, apply_act):
    grid = (N // ROW_BLK,)
    return pl.pallas_call(
        functools.partial(_out_body, apply_act),
        grid=grid,
        in_specs=[
            pl.BlockSpec((ROW_BLK, D), lambda i: (i, 0)),
            pl.BlockSpec((ROW_BLK, D), lambda i: (i, 0)),
            pl.BlockSpec((D, D), lambda i: (0, 0)),
            pl.BlockSpec((D, D), lambda i: (0, 0)),
            pl.BlockSpec((1, D), lambda i: (0, 0)),
        ],
        out_specs=pl.BlockSpec((ROW_BLK, D), lambda i: (i, 0)),
        out_shape=jax.ShapeDtypeStruct((N, D), jnp.float32),
    )(h, agg_pad, W_self, W_neigh, b_self.reshape(1, D))


def kernel(x, edge_index,
           W_pool0, b_pool0, W_neigh0, W_self0, b_self0,
           W_pool1, b_pool1, W_neigh1, W_self1, b_self1,
           W_pool2, b_pool2, W_neigh2, W_self2, b_self2):
    src = edge_index[0]
    dst = edge_index[1]
    dstp = jnp.concatenate(
        [dst, jnp.full((EP - E,), PAD_DST, jnp.int32)]).reshape(EROWS, 128)
    srcp = jnp.concatenate(
        [src, jnp.zeros((EP - E,), jnp.int32)]).reshape(EROWS, 128)
    pos, comb, spos, meta = _rank(dstp, srcp)
    ecomb = _scatter_kernel()(pos.reshape(NW, SROWS * 128),
                              comb.reshape(NW, SROWS * 128), spos)
    params = [
        (W_pool0, b_pool0, W_neigh0, W_self0, b_self0, True),
        (W_pool1, b_pool1, W_neigh1, W_self1, b_self1, True),
        (W_pool2, b_pool2, W_neigh2, W_self2, b_self2, False),
    ]
    h = x
    for (Wp, bp, Wn, Ws, bs, act) in params:
        hp = _pool_matmul(h, Wp, bp)
        agg_pad = _segmax_kernel()(hp, ecomb, meta)
        h = _out_matmul(h, agg_pad, Ws, Wn, bs, act)
    return h


# speedup vs baseline: 2.1699x; 2.1699x over previous
import functools

import jax
import jax.numpy as jnp
from jax import lax
from jax.experimental import pallas as pl
from jax.experimental.pallas import tpu as pltpu
from jax.experimental.pallas import tpu_sc as plsc

N = 10000
E = 320000
D = 128
NC, NS = 2, 16
NW = NC * NS
RANGE = 320
NPAD = NW * RANGE
SENT = RANGE
ACC_ROWS = RANGE + 8
EB = 128
EP = NW * 10240
EROWS = EP // 128
SROWS = 80
PAD_DST = 10560
OUT_SZ = E + NW * EB + EB + SROWS * 128
SZ_SP = 335872
TSLICE = SZ_SP // 16
ZB = TSLICE // 4
DUMP = E + NW * EB
ROW_BLK = 1000

@functools.cache
def _mesh():
    return plsc.VectorSubcoreMesh(
        core_axis_name="c", subcore_axis_name="s", num_cores=NC, num_subcores=NS)


def _wid():
    return lax.axis_index("s") * NC + lax.axis_index("c")


def _rank_a_body(dst_ref, src_ref, rank_ref, rc_ref, dloc_ref):
    k = pl.program_id(0)
    dst = dst_ref[...]
    bucket = dst // RANGE
    lane32 = jax.lax.broadcasted_iota(jnp.int32, (1, 32), 1)
    m = (bucket == k)
    mf = jnp.where(m, 1.0, 0.0).astype(jnp.float32)
    io0 = jax.lax.broadcasted_iota(jnp.int32, (128, 128), 0)
    io1 = jax.lax.broadcasted_iota(jnp.int32, (128, 128), 1)
    tri = jnp.where(io0 <= io1, 1.0, 0.0).astype(jnp.float32)
    c = jnp.dot(mf, tri, preferred_element_type=jnp.float32)

    @pl.when(k == 0)
    def _():
        rank_ref[...] = jnp.zeros((EROWS, 128), jnp.float32)
        rc_ref[...] = jnp.zeros((EROWS, 32), jnp.float32)
        dloc_ref[...] = (dst - bucket * RANGE) * 16384 + src_ref[...] + 1

    rank_ref[...] += jnp.where(m, c - mf, 0.0)
    rc_ref[...] += jnp.where(lane32 == k, c[:, 127:128], 0.0)


def _rank_b_body(rc_ref, exrow_ref, base_ref, spos_ref, meta_ref):
    rc_all = rc_ref[...]
    lane = jax.lax.broadcasted_iota(jnp.int32, (1, 128), 1)
    rc_cum = rc_all
    s = 1
    while s < EROWS:
        rc_cum = rc_cum + jnp.concatenate(
            [jnp.zeros((s, 32), jnp.float32), rc_cum[:-s, :]], axis=0)
        s *= 2
    totals = rc_cum[EROWS - 1:EROWS, :]
    tot_i = totals.astype(jnp.int32)
    nblk = (tot_i + (EB - 1)) // EB
    padded = nblk * EB
    base = padded
    for s in (1, 2, 4, 8, 16):
        base = base + jnp.concatenate(
            [jnp.zeros((1, s), jnp.int32), base[:, :-s]], axis=1)
    base = base - padded
    exrow_ref[...] = rc_cum - rc_all + base.astype(jnp.float32)
    base_ref[...] = base

    srows = []
    mrows = []
    zl14 = jnp.zeros((1, 14), jnp.int32)
    for k in range(NW):
        bk = base[:, k:k + 1]
        tk = tot_i[:, k:k + 1]
        pk = padded[:, k:k + 1]
        sp = jnp.where(lane < pk - tk, bk + tk + lane, DUMP + lane)
        srows.append(sp)
        mrows.append(jnp.concatenate([bk, nblk[:, k:k + 1], zl14], axis=1))
    spos_ref[...] = jnp.concatenate(srows, axis=0)
    meta_ref[...] = jnp.concatenate(mrows, axis=0)


def _rank_c_body(dst_ref, rank_ref, exrow_ref, pos_ref, acc_ref):
    k = pl.program_id(0)
    dst = dst_ref[...]
    bucket = dst // RANGE

    @pl.when(k == 0)
    def _():
        acc_ref[...] = rank_ref[...]

    lane32 = jax.lax.broadcasted_iota(jnp.int32, (1, 32), 1)
    exsel = jnp.sum(jnp.where(lane32 == k, exrow_ref[...], 0.0),
                    axis=1, keepdims=True)
    acc_ref[...] += jnp.where(bucket == k, exsel, 0.0)

    @pl.when(k == NW - 1)
    def _():
        lane = jax.lax.broadcasted_iota(jnp.int32, (1, 128), 1)
        pos = acc_ref[...].astype(jnp.int32)
        pos_ref[...] = jnp.where(dst >= N, DUMP + (lane % EB), pos)


def _rank(dstp, srcp):
    rank, rc_all, dloc = pl.pallas_call(
        _rank_a_body,
        grid=(NW,),
        in_specs=[
            pl.BlockSpec((EROWS, 128), lambda k: (0, 0)),
            pl.BlockSpec((EROWS, 128), lambda k: (0, 0)),
        ],
        out_specs=[
            pl.BlockSpec((EROWS, 128), lambda k: (0, 0)),
            pl.BlockSpec((EROWS, 32), lambda k: (0, 0)),
            pl.BlockSpec((EROWS, 128), lambda k: (0, 0)),
        ],
        out_shape=[
            jax.ShapeDtypeStruct((EROWS, 128), jnp.float32),
            jax.ShapeDtypeStruct((EROWS, 32), jnp.float32),
            jax.ShapeDtypeStruct((EROWS, 128), jnp.int32),
        ],
    )(dstp, srcp)
    exrow, base, spos, meta = pl.pallas_call(
        _rank_b_body,
        grid=(1,),
        in_specs=[pl.BlockSpec((EROWS, 32), lambda i: (0, 0))],
        out_specs=[
            pl.BlockSpec((EROWS, 32), lambda i: (0, 0)),
            pl.BlockSpec((1, 32), lambda i: (0, 0)),
            pl.BlockSpec((NW, 128), lambda i: (0, 0)),
            pl.BlockSpec((NW, 16), lambda i: (0, 0)),
        ],
        out_shape=[
            jax.ShapeDtypeStruct((EROWS, 32), jnp.float32),
            jax.ShapeDtypeStruct((1, 32), jnp.int32),
            jax.ShapeDtypeStruct((NW, 128), jnp.int32),
            jax.ShapeDtypeStruct((NW, 16), jnp.int32),
        ],
    )(rc_all)
    pos = pl.pallas_call(
        _rank_c_body,
        grid=(NW,),
        in_specs=[
            pl.BlockSpec((EROWS, 128), lambda k: (0, 0)),
            pl.BlockSpec((EROWS, 128), lambda k: (0, 0)),
            pl.BlockSpec((EROWS, 32), lambda k: (0, 0)),
        ],
        out_specs=pl.BlockSpec((EROWS, 128), lambda k: (0, 0)),
        out_shape=jax.ShapeDtypeStruct((EROWS, 128), jnp.int32),
        scratch_shapes=[pltpu.VMEM((EROWS, 128), jnp.float32)],
    )(dstp, rank, exrow)
    return pos, dloc, spos, meta


def _scatter_body(pos2, vcomb2, spos_hbm, bufa_hbm, bufb_hbm,
                  idxv, valv, sposv, dlbuf, zb, shared, sem):
    wid = _wid()
    sid = lax.axis_index("s")
    core = lax.axis_index("c")
    zero16 = jnp.zeros((16,), jnp.int32)
    for j in range(ZB // 16):
        zb[pl.ds(j * 16, 16)] = zero16
    for r in range(TSLICE // ZB):
        pltpu.sync_copy(zb, shared.at[pl.ds(sid * TSLICE + r * ZB, ZB)])
    pltpu.sync_copy(pos2.at[wid], idxv)
    pltpu.sync_copy(vcomb2.at[wid], valv)
    sent16 = jnp.zeros((16,), jnp.int32) + (SENT * 16384 + 1)
    for j in range(8):
        dlbuf[pl.ds(j * 16, 16)] = sent16
    pltpu.sync_copy(spos_hbm.at[wid], sposv)
    plsc.subcore_barrier()

    cp1 = pltpu.make_async_copy(valv, shared.at[idxv], sem)
    cp1.start()
    cp2 = pltpu.make_async_copy(dlbuf, shared.at[sposv], sem)
    cp2.start()
    cp1.wait()
    cp2.wait()
    plsc.subcore_barrier()

    sl = pl.ds(sid * TSLICE, TSLICE)

    @pl.when(core == 0)
    def _():
        pltpu.sync_copy(shared.at[sl], bufa_hbm.at[sl])

    @pl.when(core == 1)
    def _():
        pltpu.sync_copy(shared.at[sl], bufb_hbm.at[sl])


@functools.cache
def _scatter_kernel():
  return functools.partial(
    pl.kernel,
    out_type=[
        jax.ShapeDtypeStruct((SZ_SP,), jnp.int32),
        jax.ShapeDtypeStruct((SZ_SP,), jnp.int32),
    ],
    mesh=_mesh(),
    scratch_types=[
        pltpu.VMEM((SROWS * 128,), jnp.int32),
        pltpu.VMEM((SROWS * 128,), jnp.int32),
        pltpu.VMEM((128,), jnp.int32),
        pltpu.VMEM((128,), jnp.int32),
        pltpu.VMEM((ZB,), jnp.int32),
        pltpu.VMEM_SHARED((SZ_SP,), jnp.int32),
        pltpu.SemaphoreType.DMA,
    ],
  )(_scatter_body)


def _merge_body(a_ref, b_ref, o_ref):
    o_ref[...] = a_ref[...] + b_ref[...]


def _merge(a, b):
    rows = SZ_SP // 128
    out = pl.pallas_call(
        _merge_body,
        grid=(1,),
        in_specs=[
            pl.BlockSpec((rows, 128), lambda i: (0, 0)),
            pl.BlockSpec((rows, 128), lambda i: (0, 0)),
        ],
        out_specs=pl.BlockSpec((rows, 128), lambda i: (0, 0)),
        out_shape=jax.ShapeDtypeStruct((rows, 128), jnp.int32),
    )(a.reshape(rows, 128), b.reshape(rows, 128))
    return out.reshape(SZ_SP)


def _segmax_body(hp_hbm, ecomb_hbm, meta_hbm, agg_hbm,
                 acc, rows, ev, sidx, cbuf, sem0, sem1):
    wid = _wid()
    zero16 = jnp.zeros((16,), jnp.float32)

    def zrow(r, _):
        for f in range(8):
            acc[r, pl.ds(f * 16, 16)] = zero16
        return 0

    lax.fori_loop(0, ACC_ROWS, zrow, 0)

    pltpu.sync_copy(meta_hbm.at[wid], cbuf)
    cv = cbuf[pl.ds(0, 16)]
    base = cv[0]
    nblk = cv[1]

    def start_gather(b, slot, sem):
        pltpu.make_async_copy(
            hp_hbm.at[sidx.at[pl.ds(b * EB, EB)]], rows.at[slot], sem).start()

    def wait_gather(b, slot, sem):
        pltpu.make_async_copy(
            hp_hbm.at[sidx.at[pl.ds(b * EB, EB)]], rows.at[slot], sem).wait()

    def process(b, slot):
        def grp(g, _):
            dvec = (ev[pl.ds(b * EB + g * 16, 16)] - 1) >> 14
            for j in range(16):
                e = g * 16 + j
                d = dvec[j]
                for f in range(8):
                    sl = pl.ds(f * 16, 16)
                    acc[d, sl] = jnp.maximum(acc[d, sl], rows[slot, e, sl])
            return 0

        lax.fori_loop(0, EB // 16, grp, 0)

    nsb = (nblk + SROWS - 1) // SROWS

    def sblk(sb, _):
        sb_base = pl.multiple_of(base + sb * (SROWS * EB), 8)
        pltpu.sync_copy(ecomb_hbm.at[pl.ds(sb_base, SROWS * EB)], ev)

        def mkidx(r, _):
            for g in range(8):
                o = pl.ds(r * EB + g * 16, 16)
                sidx[o] = (ev[o] - 1) & 16383
            return 0

        lax.fori_loop(0, SROWS, mkidx, 0)
        lim = jnp.minimum(SROWS, nblk - sb * SROWS)
        start_gather(0, 0, sem0)

        def blk(b, _):
            @pl.when(b % 2 == 0)
            def _():
                @pl.when(b + 1 < lim)
                def _():
                    start_gather(b + 1, 1, sem1)
                wait_gather(b, 0, sem0)
                process(b, 0)

            @pl.when(b % 2 == 1)
            def _():
                @pl.when(b + 1 < lim)
                def _():
                    start_gather(b + 1, 0, sem0)
                wait_gather(b, 1, sem1)
                process(b, 1)

            return 0

        lax.fori_loop(0, lim, blk, 0)
        return 0

    lax.fori_loop(0, nsb, sblk, 0)

    pltpu.sync_copy(acc.at[pl.ds(0, RANGE)], agg_hbm.at[pl.ds(wid * RANGE, RANGE)])


@functools.cache
def _segmax_kernel():
  return functools.partial(
    pl.kernel,
    out_type=jax.ShapeDtypeStruct((NPAD, D), jnp.float32),
    mesh=_mesh(),
    scratch_types=[
        pltpu.VMEM((ACC_ROWS, D), jnp.float32),
        pltpu.VMEM((2, EB, D), jnp.float32),
        pltpu.VMEM((SROWS * EB,), jnp.int32),
        pltpu.VMEM((SROWS * EB,), jnp.int32),
        pltpu.VMEM((16,), jnp.int32),
        pltpu.SemaphoreType.DMA,
        pltpu.SemaphoreType.DMA,
    ],
  )(_segmax_body)


def _pool_body(h_ref, w_ref, b_ref, o_ref):
    acc = jnp.dot(h_ref[...], w_ref[...], preferred_element_type=jnp.float32)
    o_ref[...] = jnp.maximum(acc + b_ref[...], 0.0)


def _pool_matmul(h, W, b):
    grid = (N // ROW_BLK,)
    return pl.pallas_call(
        _pool_body,
        grid=grid,
        in_specs=[
            pl.BlockSpec((ROW_BLK, D), lambda i: (i, 0)),
            pl.BlockSpec((D, D), lambda i: (0, 0)),
            pl.BlockSpec((1, D), lambda i: (0, 0)),
        ],
        out_specs=pl.BlockSpec((ROW_BLK, D), lambda i: (i, 0)),
        out_shape=jax.ShapeDtypeStruct((N, D), jnp.float32),
    )(h, W, b.reshape(1, D))


def _out_body(apply_act, h_ref, agg_ref, ws_ref, wn_ref, b_ref, o_ref):
    acc = jnp.dot(h_ref[...], ws_ref[...], preferred_element_type=jnp.float32)
    acc += jnp.dot(agg_ref[...], wn_ref[...], preferred_element_type=jnp.float32)
    acc += b_ref[...]
    if apply_act:
        acc = jnp.where(acc > 0.0, acc, jnp.exp(jnp.minimum(acc, 0.0)) - 1.0)
    o_ref[...] = acc


def _out_matmul(h, agg_pad, W_self, W_neigh, b_self, apply_act):
    grid = (N // ROW_BLK,)
    return pl.pallas_call(
        functools.partial(_out_body, apply_act),
        grid=grid,
        in_specs=[
            pl.BlockSpec((ROW_BLK, D), lambda i: (i, 0)),
            pl.BlockSpec((ROW_BLK, D), lambda i: (i, 0)),
            pl.BlockSpec((D, D), lambda i: (0, 0)),
            pl.BlockSpec((D, D), lambda i: (0, 0)),
            pl.BlockSpec((1, D), lambda i: (0, 0)),
        ],
        out_specs=pl.BlockSpec((ROW_BLK, D), lambda i: (i, 0)),
        out_shape=jax.ShapeDtypeStruct((N, D), jnp.float32),
    )(h, agg_pad, W_self, W_neigh, b_self.reshape(1, D))


def kernel(x, edge_index,
           W_pool0, b_pool0, W_neigh0, W_self0, b_self0,
           W_pool1, b_pool1, W_neigh1, W_self1, b_self1,
           W_pool2, b_pool2, W_neigh2, W_self2, b_self2):
    src = edge_index[0]
    dst = edge_index[1]
    dstp = jnp.concatenate(
        [dst, jnp.full((EP - E,), PAD_DST, jnp.int32)]).reshape(EROWS, 128)
    srcp = jnp.concatenate(
        [src, jnp.zeros((EP - E,), jnp.int32)]).reshape(EROWS, 128)
    pos, comb, spos, meta = _rank(dstp, srcp)
    bufa, bufb = _scatter_kernel()(pos.reshape(NW, SROWS * 128),
                                   comb.reshape(NW, SROWS * 128), spos)
    ecomb = _merge(bufa, bufb)
    params = [
        (W_pool0, b_pool0, W_neigh0, W_self0, b_self0, True),
        (W_pool1, b_pool1, W_neigh1, W_self1, b_self1, True),
        (W_pool2, b_pool2, W_neigh2, W_self2, b_self2, False),
    ]
    h = x
    for (Wp, bp, Wn, Ws, bs, act) in params:
        hp = _pool_matmul(h, Wp, bp)
        agg_pad = _segmax_kernel()(hp, ecomb, meta)
        h = _out_matmul(h, agg_pad, Ws, Wn, bs, act)
    return h
